# Initial kernel scaffold; baseline (speedup 1.0000x reference)
import jax
import jax.numpy as jnp
from jax.experimental import pallas as pl


def kernel(timesteps, table):
    raise NotImplementedError("write your pallas kernel here")


# speedup vs baseline: 1.6298x; 1.6298x over previous
import functools

import jax
import jax.numpy as jnp
from jax import lax
from jax.experimental import pallas as pl
from jax.experimental.pallas import tpu as pltpu
from jax.experimental.pallas import tpu_sc as plsc

_NC = 2
_NS = 16
_NW = _NC * _NS


@functools.partial(jax.jit, static_argnums=(2, 3, 4))
def _sc_gather(table, idx, b_per_w, nchunks, chunk):
    vocab, d = table.shape
    b_total = _NW * b_per_w
    mesh = plsc.VectorSubcoreMesh(core_axis_name="c", subcore_axis_name="s")

    @functools.partial(
        pl.kernel,
        mesh=mesh,
        out_type=jax.ShapeDtypeStruct((b_total, d), jnp.float32),
        scratch_types=[
            pltpu.VMEM((nchunks, chunk), jnp.int32),
            pltpu.VMEM((chunk, d), jnp.float32),
            pltpu.VMEM((chunk, d), jnp.float32),
            pltpu.SemaphoreType.DMA,
            pltpu.SemaphoreType.DMA,
        ],
    )
    def k(table_hbm, idx_hbm, out_hbm, idx_v, buf0, buf1, sem0, sem1):
        wid = lax.axis_index("s") * _NC + lax.axis_index("c")
        base = wid * b_per_w
        pltpu.sync_copy(idx_hbm.at[wid], idx_v)
        pltpu.async_copy(table_hbm.at[idx_v.at[0]], buf0, sem0)

        @pl.loop(0, nchunks // 2)
        def _(t):
            c0 = 2 * t
            pltpu.make_async_copy(table_hbm.at[idx_v.at[c0]], buf0, sem0).wait()
            pltpu.async_copy(table_hbm.at[idx_v.at[c0 + 1]], buf1, sem1)
            pltpu.sync_copy(buf0, out_hbm.at[pl.ds(base + c0 * chunk, chunk)])
            pltpu.make_async_copy(
                table_hbm.at[idx_v.at[c0 + 1]], buf1, sem1).wait()

            @pl.when(t + 1 < nchunks // 2)
            def _():
                pltpu.async_copy(table_hbm.at[idx_v.at[c0 + 2]], buf0, sem0)

            pltpu.sync_copy(
                buf1, out_hbm.at[pl.ds(base + (c0 + 1) * chunk, chunk)])

    return k(table, idx)


def kernel(timesteps, table):
    b, l, _ = timesteps.shape
    d = table.shape[1]
    n = b * l
    b_per_w = n // _NW
    chunk = 80
    nchunks = b_per_w // chunk
    idx = timesteps.astype(jnp.int32).reshape(_NW, nchunks, chunk)
    out = _sc_gather(table, idx, b_per_w, nchunks, chunk)
    return out.reshape(b, l, d)
